# baseline (speedup 1.0000x reference)
import functools

import jax
import jax.numpy as jnp
from jax import lax
from jax.experimental import pallas as pl
from jax.experimental.pallas import tpu as pltpu
from jax.experimental.pallas import tpu_sc as plsc

N = 10000
E = 160000
D = 256
G = 32
EPS = 1e-5

N_PAD = 10240
PAD_ROWS = N_PAD - N
E_PAD = 163840
CHUNK = 128
DEG_CHUNKS = E_PAD // (32 * CHUNK)
ACHUNK = 64
AGG_CHUNKS = E_PAD // (16 * ACHUNK)
AGG_QTR = AGG_CHUNKS // 4
NBUF = 4
FEAT = 128
ROWS_PER_SUB = N_PAD // 16

def _deg_body(dst_hbm, zeros_hbm, out_hbm, idx_v, ones_v, deg_sp):
    c = lax.axis_index("c")
    s = lax.axis_index("s")
    wid = s * 2 + c
    pltpu.sync_copy(zeros_hbm.at[pl.ds(s * ROWS_PER_SUB, ROWS_PER_SUB)],
                    deg_sp.at[pl.ds(s * ROWS_PER_SUB, ROWS_PER_SUB)])
    for i in range(CHUNK // 16):
        ones_v[pl.ds(i * 16, 16)] = jnp.ones((16,), jnp.float32)
    pltpu.sync_copy(dst_hbm.at[wid], idx_v)
    plsc.subcore_barrier()

    def step(j, _):
        pltpu.sync_copy(ones_v, deg_sp.at[idx_v.at[j]], add=True)
        return _

    lax.fori_loop(0, DEG_CHUNKS, step, None)
    plsc.subcore_barrier()
    pltpu.sync_copy(deg_sp.at[pl.ds(s * ROWS_PER_SUB, ROWS_PER_SUB)],
                    out_hbm.at[c, pl.ds(s * ROWS_PER_SUB, ROWS_PER_SUB)])


@functools.cache
def _sc_mesh():
    return plsc.VectorSubcoreMesh(core_axis_name="c", subcore_axis_name="s",
                                  num_cores=2, num_subcores=16)


@functools.cache
def _make_deg_kernel():
    return pl.kernel(
        _deg_body,
        out_type=jax.ShapeDtypeStruct((2, N_PAD), jnp.float32),
        mesh=_sc_mesh(),
        scratch_types=[
            pltpu.VMEM((DEG_CHUNKS, CHUNK), jnp.int32),
            pltpu.VMEM((CHUNK,), jnp.float32),
            pltpu.VMEM_SHARED((N_PAD,), jnp.float32),
        ],
    )


def _deg_kernel(dst_deg, zeros_hbm):
    return _make_deg_kernel()(dst_deg, zeros_hbm)


def _agg_body(h_hbm, src_hbm, dst_hbm, out_hbm, src_v, dst_v, rows0, rows1,
              rows2, rows3, sem0, sem1, sem2, sem3, agg_sp):
    c = lax.axis_index("c")
    s = lax.axis_index("s")
    rows = (rows0, rows1, rows2, rows3)
    sems = (sem0, sem1, sem2, sem3)
    base = c * N_PAD + s * ROWS_PER_SUB
    pltpu.sync_copy(h_hbm.at[pl.ds(base, ROWS_PER_SUB)],
                    agg_sp.at[pl.ds(s * ROWS_PER_SUB, ROWS_PER_SUB)])
    plsc.subcore_barrier()

    def _gather(j, buf, sem):
        return pltpu.async_copy(h_hbm.at[src_v.at[j]], buf, sem)

    def _drain(j, buf, sem):
        pltpu.make_async_copy(h_hbm.at[src_v.at[j]], buf, sem).wait()

    for qtr in range(4):
        pltpu.sync_copy(src_hbm.at[c, s, pl.ds(qtr * AGG_QTR, AGG_QTR)],
                        src_v)
        pltpu.sync_copy(dst_hbm.at[s, pl.ds(qtr * AGG_QTR, AGG_QTR)],
                        dst_v)
        for b in range(NBUF):
            _gather(b, rows[b], sems[b])

        def quad(t, _):
            for b in range(NBUF):
                j = t * NBUF + b
                _drain(j, rows[b], sems[b])
                pltpu.sync_copy(rows[b], agg_sp.at[dst_v.at[j]], add=True)
                _gather(j + NBUF, rows[b], sems[b])
            return _

        lax.fori_loop(0, AGG_QTR // NBUF - 1, quad, None)
        for b in range(NBUF):
            j = AGG_QTR - NBUF + b
            _drain(j, rows[b], sems[b])
            pltpu.sync_copy(rows[b], agg_sp.at[dst_v.at[j]], add=True)

    plsc.subcore_barrier()
    pltpu.sync_copy(agg_sp.at[pl.ds(s * ROWS_PER_SUB, ROWS_PER_SUB)],
                    out_hbm.at[c, pl.ds(s * ROWS_PER_SUB, ROWS_PER_SUB)])


@functools.cache
def _make_agg_kernel():
    return pl.kernel(
        _agg_body,
        out_type=jax.ShapeDtypeStruct((2, N_PAD, FEAT), jnp.float32),
        mesh=_sc_mesh(),
        scratch_types=[
            pltpu.VMEM((AGG_QTR, ACHUNK), jnp.int32),
            pltpu.VMEM((AGG_QTR, ACHUNK), jnp.int32),
            pltpu.VMEM((ACHUNK, FEAT), jnp.float32),
            pltpu.VMEM((ACHUNK, FEAT), jnp.float32),
            pltpu.VMEM((ACHUNK, FEAT), jnp.float32),
            pltpu.VMEM((ACHUNK, FEAT), jnp.float32),
            pltpu.SemaphoreType.DMA,
            pltpu.SemaphoreType.DMA,
            pltpu.SemaphoreType.DMA,
            pltpu.SemaphoreType.DMA,
            pltpu.VMEM_SHARED((N_PAD, FEAT), jnp.float32),
        ],
    )


def _agg_kernel(h_flat, src3, dst3):
    return _make_agg_kernel()(h_flat, src3, dst3)


_LBLK = 2048


def _lin_body(x_ref, w_ref, degA_ref, degB_ref, h_ref, dinv_ref):
    deg = degA_ref[0, 0] + degB_ref[0, 0] + 1.0
    dinv = lax.rsqrt(deg)
    h = jnp.dot(x_ref[...], w_ref[...], preferred_element_type=jnp.float32)
    h = h * dinv[:, None]
    h_ref[0] = h[:, :FEAT]
    h_ref[1] = h[:, FEAT:]
    dinv_ref[0, 0] = dinv


def _lin(x_pad, W, degA, degB):
    nb = N_PAD // _LBLK
    return pl.pallas_call(
        _lin_body,
        grid=(nb,),
        in_specs=[
            pl.BlockSpec((_LBLK, D), lambda i: (i, 0)),
            pl.BlockSpec((D, D), lambda i: (0, 0)),
            pl.BlockSpec((1, 1, _LBLK), lambda i: (i, 0, 0)),
            pl.BlockSpec((1, 1, _LBLK), lambda i: (i, 0, 0)),
        ],
        out_specs=[
            pl.BlockSpec((2, _LBLK, FEAT), lambda i: (0, i, 0)),
            pl.BlockSpec((1, 1, _LBLK), lambda i: (i, 0, 0)),
        ],
        out_shape=[
            jax.ShapeDtypeStruct((2, N_PAD, FEAT), jnp.float32),
            jax.ShapeDtypeStruct((nb, 1, _LBLK), jnp.float32),
        ],
    )(x_pad, W, degA, degB)


_ABLK = 2048
_BBLK = 2000
_NBA = N_PAD // _ABLK
_NBB = N // _BBLK


def _fused_body(agg_ref, dinv_ref, b_ref, batchA_ref, batchB_ref, w_ref,
                bias_ref, a_ref, x_ref, out_ref, gcn_s, s1_s, s2_s, cnt_s):
    i = pl.program_id(0)

    @pl.when(i < _NBA)
    def _phase_a():
        a = agg_ref[...]
        gcn = jnp.concatenate([a[0], a[1]], axis=1)
        gcn = gcn * dinv_ref[0, 0][:, None] + b_ref[...]
        gcn_s[pl.ds(i * _ABLK, _ABLK), :] = gcn
        bt = batchA_ref[0, 0]
        oh = (lax.broadcasted_iota(jnp.int32, (G, _ABLK), 0)
              == bt[None, :]).astype(jnp.float32)
        s1 = jnp.dot(oh, gcn, preferred_element_type=jnp.float32)
        s2 = jnp.dot(oh, gcn * gcn, preferred_element_type=jnp.float32)
        cn = jnp.dot(oh, jnp.ones((_ABLK, 128), jnp.float32),
                     preferred_element_type=jnp.float32)
        @pl.when(i == 0)
        def _():
            s1_s[...] = s1
            s2_s[...] = s2
            cnt_s[...] = cn

        @pl.when(i > 0)
        def _():
            s1_s[...] += s1
            s2_s[...] += s2
            cnt_s[...] += cn

    @pl.when(i >= _NBA)
    def _phase_b():
        cnt = jnp.maximum(cnt_s[...][:, :1], 1.0)
        inv_cnt = 1.0 / cnt
        m = s1_s[...] * inv_cnt
        q = s2_s[...] * inv_cnt
        a = a_ref[...]
        var = q + m * m * (a * a - 2.0 * a)
        inv_std = lax.rsqrt(var + EPS)
        m_scaled = m * a
        bt = batchB_ref[0, 0]
        oh = (bt[:, None] == lax.broadcasted_iota(
            jnp.int32, (_BBLK, G), 1)).astype(jnp.float32)
        mean_rows = jnp.dot(oh, m_scaled, preferred_element_type=jnp.float32)
        scale_rows = jnp.dot(oh, inv_std, preferred_element_type=jnp.float32)
        gcn = gcn_s[pl.ds((i - _NBA) * _BBLK, _BBLK), :]
        y = (gcn - mean_rows) * scale_rows * w_ref[...] + bias_ref[...]
        out_ref[...] = jnp.maximum(y, 0.0) + x_ref[...]


def _fused_norm(agg, dinv3, b2, batch_pad3, batch3, w2, bias2, a2, x):
    return pl.pallas_call(
        _fused_body,
        grid=(_NBA + _NBB,),
        in_specs=[
            pl.BlockSpec((2, _ABLK, FEAT),
                         lambda i: (0, jnp.where(i < _NBA, i, 0), 0)),
            pl.BlockSpec((1, 1, _ABLK),
                         lambda i: (jnp.where(i < _NBA, i, 0), 0, 0)),
            pl.BlockSpec((1, D), lambda i: (0, 0)),
            pl.BlockSpec((1, 1, _ABLK),
                         lambda i: (jnp.where(i < _NBA, i, 0), 0, 0)),
            pl.BlockSpec((1, 1, _BBLK),
                         lambda i: (jnp.where(i < _NBA, 0, i - _NBA), 0, 0)),
            pl.BlockSpec((1, D), lambda i: (0, 0)),
            pl.BlockSpec((1, D), lambda i: (0, 0)),
            pl.BlockSpec((1, D), lambda i: (0, 0)),
            pl.BlockSpec((_BBLK, D),
                         lambda i: (jnp.where(i < _NBA, 0, i - _NBA), 0)),
        ],
        out_specs=pl.BlockSpec(
            (_BBLK, D), lambda i: (jnp.where(i < _NBA, 0, i - _NBA), 0)),
        out_shape=jax.ShapeDtypeStruct((N, D), jnp.float32),
        scratch_shapes=[
            pltpu.VMEM((N_PAD, D), jnp.float32),
            pltpu.VMEM((G, D), jnp.float32),
            pltpu.VMEM((G, D), jnp.float32),
            pltpu.VMEM((G, 128), jnp.float32),
        ],
    )(agg, dinv3, b2, batch_pad3, batch3, w2, bias2, a2, x)


def kernel(x, edge_index, batch, W, b, gn_weight, gn_bias, gn_mean_scale):
    f32 = jnp.float32
    pad_i = jnp.arange(E_PAD - E, dtype=jnp.int32)
    pad_row = N + pad_i % PAD_ROWS
    src_full = jnp.concatenate([edge_index[0], pad_row])
    dst_full = jnp.concatenate([edge_index[1], pad_row])
    src3 = jnp.stack([src_full, src_full + N_PAD]).reshape(
        2, 16, AGG_CHUNKS, ACHUNK)
    dst3 = dst_full.reshape(16, AGG_CHUNKS, ACHUNK)
    dst_deg = dst_full.reshape(32, DEG_CHUNKS, CHUNK)
    x_pad = jnp.zeros((N_PAD, D), f32).at[:N].set(x)
    zeros_hbm = jnp.zeros((N_PAD,), f32)
    batch_pad3 = jnp.concatenate(
        [batch, jnp.full((PAD_ROWS,), G, jnp.int32)]).reshape(
            N_PAD // _ABLK, 1, _ABLK)
    batch3 = batch.reshape(N // _BBLK, 1, _BBLK)
    b2 = b.reshape(1, D)
    w2 = gn_weight.reshape(1, D)
    bias2 = gn_bias.reshape(1, D)
    a2 = gn_mean_scale.reshape(1, D)

    deg = _deg_kernel(dst_deg, zeros_hbm)
    degA = deg[0].reshape(N_PAD // _LBLK, 1, _LBLK)
    degB = deg[1].reshape(N_PAD // _LBLK, 1, _LBLK)
    h_stack, dinv3 = _lin(x_pad, W, degA, degB)
    h_flat = h_stack.reshape(2 * N_PAD, FEAT)
    agg = _agg_kernel(h_flat, src3, dst3)
    return _fused_norm(agg, dinv3, b2, batch_pad3, batch3, w2, bias2, a2, x)


# speedup vs baseline: 1.0060x; 1.0060x over previous
import functools

import jax
import jax.numpy as jnp
from jax import lax
from jax.experimental import pallas as pl
from jax.experimental.pallas import tpu as pltpu
from jax.experimental.pallas import tpu_sc as plsc

N = 10000
E = 160000
D = 256
G = 32
EPS = 1e-5

N_PAD = 10240
PAD_ROWS = N_PAD - N
E_PAD = 163840
CHUNK = 128
DEG_CHUNKS = E_PAD // (32 * CHUNK)
AGG_CHUNKS = E_PAD // (16 * CHUNK)
AGG_HALF = AGG_CHUNKS // 2
FEAT = 128
ROWS_PER_SUB = N_PAD // 16

def _deg_body(dst_hbm, zeros_hbm, out_hbm, idx_v, ones_v, deg_sp):
    c = lax.axis_index("c")
    s = lax.axis_index("s")
    wid = s * 2 + c
    pltpu.sync_copy(zeros_hbm.at[pl.ds(s * ROWS_PER_SUB, ROWS_PER_SUB)],
                    deg_sp.at[pl.ds(s * ROWS_PER_SUB, ROWS_PER_SUB)])
    for i in range(CHUNK // 16):
        ones_v[pl.ds(i * 16, 16)] = jnp.ones((16,), jnp.float32)
    pltpu.sync_copy(dst_hbm.at[wid], idx_v)
    plsc.subcore_barrier()

    def step(j, _):
        pltpu.sync_copy(ones_v, deg_sp.at[idx_v.at[j]], add=True)
        return _

    lax.fori_loop(0, DEG_CHUNKS, step, None)
    plsc.subcore_barrier()
    pltpu.sync_copy(deg_sp.at[pl.ds(s * ROWS_PER_SUB, ROWS_PER_SUB)],
                    out_hbm.at[c, pl.ds(s * ROWS_PER_SUB, ROWS_PER_SUB)])


@functools.cache
def _sc_mesh():
    return plsc.VectorSubcoreMesh(core_axis_name="c", subcore_axis_name="s",
                                  num_cores=2, num_subcores=16)


@functools.cache
def _make_deg_kernel():
    return pl.kernel(
        _deg_body,
        out_type=jax.ShapeDtypeStruct((2, N_PAD), jnp.float32),
        mesh=_sc_mesh(),
        scratch_types=[
            pltpu.VMEM((DEG_CHUNKS, CHUNK), jnp.int32),
            pltpu.VMEM((CHUNK,), jnp.float32),
            pltpu.VMEM_SHARED((N_PAD,), jnp.float32),
        ],
    )


def _deg_kernel(dst_deg, zeros_hbm):
    return _make_deg_kernel()(dst_deg, zeros_hbm)


def _agg_body(h_hbm, src_hbm, dst_hbm, out_hbm, src_v, dst_v, rows0, rows1,
              sem0, sem1, agg_sp):
    c = lax.axis_index("c")
    s = lax.axis_index("s")
    base = c * N_PAD + s * ROWS_PER_SUB
    pltpu.sync_copy(h_hbm.at[pl.ds(base, ROWS_PER_SUB)],
                    agg_sp.at[pl.ds(s * ROWS_PER_SUB, ROWS_PER_SUB)])
    plsc.subcore_barrier()

    def _gather(j, buf, sem):
        return pltpu.async_copy(h_hbm.at[src_v.at[j]], buf, sem)

    def _drain(j, buf, sem):
        pltpu.make_async_copy(h_hbm.at[src_v.at[j]], buf, sem).wait()

    for half in range(2):
        pltpu.sync_copy(src_hbm.at[c, s, pl.ds(half * AGG_HALF, AGG_HALF)],
                        src_v)
        pltpu.sync_copy(dst_hbm.at[s, pl.ds(half * AGG_HALF, AGG_HALF)],
                        dst_v)
        _gather(0, rows0, sem0)

        def pair(t, _):
            j = t * 2
            _gather(j + 1, rows1, sem1)
            _drain(j, rows0, sem0)
            pltpu.sync_copy(rows0, agg_sp.at[dst_v.at[j]], add=True)
            _gather(j + 2, rows0, sem0)
            _drain(j + 1, rows1, sem1)
            pltpu.sync_copy(rows1, agg_sp.at[dst_v.at[j + 1]], add=True)
            return _

        lax.fori_loop(0, AGG_HALF // 2 - 1, pair, None)
        j = AGG_HALF - 2
        _gather(j + 1, rows1, sem1)
        _drain(j, rows0, sem0)
        pltpu.sync_copy(rows0, agg_sp.at[dst_v.at[j]], add=True)
        _drain(j + 1, rows1, sem1)
        pltpu.sync_copy(rows1, agg_sp.at[dst_v.at[j + 1]], add=True)

    plsc.subcore_barrier()
    pltpu.sync_copy(agg_sp.at[pl.ds(s * ROWS_PER_SUB, ROWS_PER_SUB)],
                    out_hbm.at[c, pl.ds(s * ROWS_PER_SUB, ROWS_PER_SUB)])


@functools.cache
def _make_agg_kernel():
    return pl.kernel(
        _agg_body,
        out_type=jax.ShapeDtypeStruct((2, N_PAD, FEAT), jnp.float32),
        mesh=_sc_mesh(),
        scratch_types=[
            pltpu.VMEM((AGG_HALF, CHUNK), jnp.int32),
            pltpu.VMEM((AGG_HALF, CHUNK), jnp.int32),
            pltpu.VMEM((CHUNK, FEAT), jnp.float32),
            pltpu.VMEM((CHUNK, FEAT), jnp.float32),
            pltpu.SemaphoreType.DMA,
            pltpu.SemaphoreType.DMA,
            pltpu.VMEM_SHARED((N_PAD, FEAT), jnp.float32),
        ],
    )


def _agg_kernel(h_flat, src3, dst3):
    return _make_agg_kernel()(h_flat, src3, dst3)


_LBLK = 2048


def _lin_body(x_ref, w_ref, degA_ref, degB_ref, h_ref, dinv_ref):
    deg = degA_ref[0, 0] + degB_ref[0, 0] + 1.0
    dinv = lax.rsqrt(deg)
    h = jnp.dot(x_ref[...], w_ref[...], preferred_element_type=jnp.float32)
    h = h * dinv[:, None]
    h_ref[0] = h[:, :FEAT]
    h_ref[1] = h[:, FEAT:]
    dinv_ref[0, 0] = dinv


def _lin(x_pad, W, degA, degB):
    nb = N_PAD // _LBLK
    return pl.pallas_call(
        _lin_body,
        grid=(nb,),
        in_specs=[
            pl.BlockSpec((_LBLK, D), lambda i: (i, 0)),
            pl.BlockSpec((D, D), lambda i: (0, 0)),
            pl.BlockSpec((1, 1, _LBLK), lambda i: (i, 0, 0)),
            pl.BlockSpec((1, 1, _LBLK), lambda i: (i, 0, 0)),
        ],
        out_specs=[
            pl.BlockSpec((2, _LBLK, FEAT), lambda i: (0, i, 0)),
            pl.BlockSpec((1, 1, _LBLK), lambda i: (i, 0, 0)),
        ],
        out_shape=[
            jax.ShapeDtypeStruct((2, N_PAD, FEAT), jnp.float32),
            jax.ShapeDtypeStruct((nb, 1, _LBLK), jnp.float32),
        ],
    )(x_pad, W, degA, degB)


_ABLK = 2048
_BBLK = 2000
_NBA = N_PAD // _ABLK
_NBB = N // _BBLK


def _fused_body(agg_ref, dinv_ref, b_ref, batchA_ref, batchB_ref, w_ref,
                bias_ref, a_ref, x_ref, out_ref, gcn_s, s1_s, s2_s, cnt_s):
    i = pl.program_id(0)

    @pl.when(i < _NBA)
    def _phase_a():
        a = agg_ref[...]
        gcn = jnp.concatenate([a[0], a[1]], axis=1)
        gcn = gcn * dinv_ref[0, 0][:, None] + b_ref[...]
        gcn_s[pl.ds(i * _ABLK, _ABLK), :] = gcn
        bt = batchA_ref[0, 0]
        oh = (lax.broadcasted_iota(jnp.int32, (G, _ABLK), 0)
              == bt[None, :]).astype(jnp.float32)
        s1 = jnp.dot(oh, gcn, preferred_element_type=jnp.float32)
        s2 = jnp.dot(oh, gcn * gcn, preferred_element_type=jnp.float32)
        cn = jnp.dot(oh, jnp.ones((_ABLK, 128), jnp.float32),
                     preferred_element_type=jnp.float32)
        @pl.when(i == 0)
        def _():
            s1_s[...] = s1
            s2_s[...] = s2
            cnt_s[...] = cn

        @pl.when(i > 0)
        def _():
            s1_s[...] += s1
            s2_s[...] += s2
            cnt_s[...] += cn

    @pl.when(i >= _NBA)
    def _phase_b():
        cnt = jnp.maximum(cnt_s[...][:, :1], 1.0)
        inv_cnt = 1.0 / cnt
        m = s1_s[...] * inv_cnt
        q = s2_s[...] * inv_cnt
        a = a_ref[...]
        var = q + m * m * (a * a - 2.0 * a)
        inv_std = lax.rsqrt(var + EPS)
        m_scaled = m * a
        bt = batchB_ref[0, 0]
        oh = (bt[:, None] == lax.broadcasted_iota(
            jnp.int32, (_BBLK, G), 1)).astype(jnp.float32)
        mean_rows = jnp.dot(oh, m_scaled, preferred_element_type=jnp.float32)
        scale_rows = jnp.dot(oh, inv_std, preferred_element_type=jnp.float32)
        gcn = gcn_s[pl.ds((i - _NBA) * _BBLK, _BBLK), :]
        y = (gcn - mean_rows) * scale_rows * w_ref[...] + bias_ref[...]
        out_ref[...] = jnp.maximum(y, 0.0) + x_ref[...]


def _fused_norm(agg, dinv3, b2, batch_pad3, batch3, w2, bias2, a2, x):
    return pl.pallas_call(
        _fused_body,
        grid=(_NBA + _NBB,),
        in_specs=[
            pl.BlockSpec((2, _ABLK, FEAT),
                         lambda i: (0, jnp.where(i < _NBA, i, 0), 0)),
            pl.BlockSpec((1, 1, _ABLK),
                         lambda i: (jnp.where(i < _NBA, i, 0), 0, 0)),
            pl.BlockSpec((1, D), lambda i: (0, 0)),
            pl.BlockSpec((1, 1, _ABLK),
                         lambda i: (jnp.where(i < _NBA, i, 0), 0, 0)),
            pl.BlockSpec((1, 1, _BBLK),
                         lambda i: (jnp.where(i < _NBA, 0, i - _NBA), 0, 0)),
            pl.BlockSpec((1, D), lambda i: (0, 0)),
            pl.BlockSpec((1, D), lambda i: (0, 0)),
            pl.BlockSpec((1, D), lambda i: (0, 0)),
            pl.BlockSpec((_BBLK, D),
                         lambda i: (jnp.where(i < _NBA, 0, i - _NBA), 0)),
        ],
        out_specs=pl.BlockSpec(
            (_BBLK, D), lambda i: (jnp.where(i < _NBA, 0, i - _NBA), 0)),
        out_shape=jax.ShapeDtypeStruct((N, D), jnp.float32),
        scratch_shapes=[
            pltpu.VMEM((N_PAD, D), jnp.float32),
            pltpu.VMEM((G, D), jnp.float32),
            pltpu.VMEM((G, D), jnp.float32),
            pltpu.VMEM((G, 128), jnp.float32),
        ],
    )(agg, dinv3, b2, batch_pad3, batch3, w2, bias2, a2, x)


def kernel(x, edge_index, batch, W, b, gn_weight, gn_bias, gn_mean_scale):
    f32 = jnp.float32
    pad_i = jnp.arange(E_PAD - E, dtype=jnp.int32)
    pad_row = N + pad_i % PAD_ROWS
    src_full = jnp.concatenate([edge_index[0], pad_row])
    dst_full = jnp.concatenate([edge_index[1], pad_row])
    src3 = jnp.stack([src_full, src_full + N_PAD]).reshape(
        2, 16, AGG_CHUNKS, CHUNK)
    dst3 = dst_full.reshape(16, AGG_CHUNKS, CHUNK)
    dst_deg = dst_full.reshape(32, DEG_CHUNKS, CHUNK)
    x_pad = jnp.zeros((N_PAD, D), f32).at[:N].set(x)
    zeros_hbm = jnp.zeros((N_PAD,), f32)
    batch_pad3 = jnp.concatenate(
        [batch, jnp.full((PAD_ROWS,), G, jnp.int32)]).reshape(
            N_PAD // _ABLK, 1, _ABLK)
    batch3 = batch.reshape(N // _BBLK, 1, _BBLK)
    b2 = b.reshape(1, D)
    w2 = gn_weight.reshape(1, D)
    bias2 = gn_bias.reshape(1, D)
    a2 = gn_mean_scale.reshape(1, D)

    deg = _deg_kernel(dst_deg, zeros_hbm)
    degA = deg[0].reshape(N_PAD // _LBLK, 1, _LBLK)
    degB = deg[1].reshape(N_PAD // _LBLK, 1, _LBLK)
    h_stack, dinv3 = _lin(x_pad, W, degA, degB)
    h_flat = h_stack.reshape(2 * N_PAD, FEAT)
    agg = _agg_kernel(h_flat, src3, dst3)
    return _fused_norm(agg, dinv3, b2, batch_pad3, batch3, w2, bias2, a2, x)
